# Initial kernel scaffold; baseline (speedup 1.0000x reference)
import jax
import jax.numpy as jnp
from jax.experimental import pallas as pl


def kernel(x, source, target, edge_type, blocks):
    raise NotImplementedError("write your pallas kernel here")


# speedup vs baseline: 4.5135x; 4.5135x over previous
import functools

import jax
import jax.numpy as jnp
from jax import lax
from jax.experimental import pallas as pl
from jax.experimental.pallas import tpu as pltpu
from jax.experimental.pallas import tpu_sc as plsc

N_NODES = 10000
N_EDGES = 320000
N_REL = 24
DIM = 128
N_BLK = 8
BLK = 16

M_TILE = 2000
M_TILES = N_NODES // M_TILE

NC = 2
NS = 16
NW = NC * NS
EPW = N_EDGES // NW
CHUNK = 80
NCHUNKS = EPW // CHUNK


def _transform_body(x_ref, blocks_ref, y_ref, w_ref):
    m = pl.program_id(0)
    r = pl.program_id(1)

    @pl.when((m == 0) & (r == 0))
    def _():
        w_ref[...] = jnp.zeros_like(w_ref)

    for b in range(N_BLK):
        w_ref[b * BLK:(b + 1) * BLK, b * BLK:(b + 1) * BLK] = blocks_ref[0, b]
    y_ref[...] = jnp.dot(x_ref[...], w_ref[...],
                         preferred_element_type=jnp.float32)


def _transform(x, blocks, *, interpret=False):
    return pl.pallas_call(
        _transform_body,
        grid=(M_TILES, N_REL),
        in_specs=[
            pl.BlockSpec((M_TILE, DIM), lambda m, r: (m, 0)),
            pl.BlockSpec((1, N_BLK, BLK, BLK), lambda m, r: (r, 0, 0, 0)),
        ],
        out_specs=pl.BlockSpec((M_TILE, DIM), lambda m, r: (r * M_TILES + m, 0)),
        out_shape=jax.ShapeDtypeStruct((N_REL * N_NODES, DIM), jnp.float32),
        scratch_shapes=[pltpu.VMEM((DIM, DIM), jnp.float32)],
        interpret=interpret,
    )(x, blocks)


def _sc_body(y_hbm, src_hbm, tgt_hbm, et_hbm, zeros_hbm, out_hbm,
             src_v, et_v, tgt_v, idx_v, rows_v, acc_sh, sem):
    cid = lax.axis_index("c")
    sid = lax.axis_index("s")
    wid = cid * NS + sid

    zrows = N_NODES // 10
    row0 = sid * zrows

    @pl.when(sid < 10)
    def _():
        pltpu.sync_copy(zeros_hbm.at[pl.ds(row0, zrows)],
                        acc_sh.at[pl.ds(row0, zrows)])

    plsc.subcore_barrier()

    base = wid * EPW

    def body(ci, _):
        off = base + ci * CHUNK
        pltpu.sync_copy(src_hbm.at[pl.ds(off, CHUNK)], src_v)
        pltpu.sync_copy(et_hbm.at[pl.ds(off, CHUNK)], et_v)
        pltpu.sync_copy(tgt_hbm.at[pl.ds(off, CHUNK)], tgt_v)
        for i in range(CHUNK // 16):
            s = pl.ds(i * 16, 16)
            idx_v[s] = et_v[s] * N_NODES + src_v[s]
        pltpu.async_copy(y_hbm.at[idx_v], rows_v, sem).wait()
        pltpu.sync_copy(rows_v, acc_sh.at[tgt_v], add=True)
        return ()

    lax.fori_loop(0, NCHUNKS, body, ())
    plsc.subcore_barrier()

    @pl.when(sid < 10)
    def _():
        pltpu.sync_copy(acc_sh.at[pl.ds(row0, zrows)],
                        out_hbm.at[cid, pl.ds(row0, zrows)])


def _sc_scatter(y, source, target, edge_type, zeros):
    mesh = plsc.VectorSubcoreMesh(core_axis_name="c", subcore_axis_name="s")
    k = functools.partial(
        pl.kernel,
        out_type=jax.ShapeDtypeStruct((NC, N_NODES, DIM), jnp.float32),
        mesh=mesh,
        scratch_types=[
            pltpu.VMEM((CHUNK,), jnp.int32),
            pltpu.VMEM((CHUNK,), jnp.int32),
            pltpu.VMEM((CHUNK,), jnp.int32),
            pltpu.VMEM((CHUNK,), jnp.int32),
            pltpu.VMEM((CHUNK, DIM), jnp.float32),
            pltpu.VMEM_SHARED((N_NODES, DIM), jnp.float32),
            pltpu.SemaphoreType.DMA,
        ],
    )(_sc_body)
    return k(y, source, target, edge_type, zeros)


def _combine_body(p_ref, o_ref):
    o_ref[...] = p_ref[0] + p_ref[1]


def _combine(partials, *, interpret=False):
    return pl.pallas_call(
        _combine_body,
        grid=(M_TILES,),
        in_specs=[pl.BlockSpec((NC, M_TILE, DIM), lambda i: (0, i, 0))],
        out_specs=pl.BlockSpec((M_TILE, DIM), lambda i: (i, 0)),
        out_shape=jax.ShapeDtypeStruct((N_NODES, DIM), jnp.float32),
        interpret=interpret,
    )(partials)


def kernel(x, source, target, edge_type, blocks):
    y = _transform(x, blocks)
    zeros = jnp.zeros((N_NODES, DIM), jnp.float32)
    partials = _sc_scatter(y, source, target, edge_type, zeros)
    return _combine(partials)


# speedup vs baseline: 8.8082x; 1.9515x over previous
import functools

import jax
import jax.numpy as jnp
from jax import lax
from jax.experimental import pallas as pl
from jax.experimental.pallas import tpu as pltpu
from jax.experimental.pallas import tpu_sc as plsc

N_NODES = 10000
N_EDGES = 320000
N_REL = 24
DIM = 128
N_BLK = 8
BLK = 16

M_TILE = 2000
M_TILES = N_NODES // M_TILE

NC = 2
NS = 16
NW = NC * NS
EPW = N_EDGES // NW
CHUNK = 80
NCHUNKS = EPW // CHUNK
NBUF = 2
MAIN = (NCHUNKS // NBUF) * NBUF


def _transform_body(x_ref, blocks_ref, y_ref, w_ref):
    m = pl.program_id(0)
    r = pl.program_id(1)

    @pl.when((m == 0) & (r == 0))
    def _():
        w_ref[...] = jnp.zeros_like(w_ref)

    for b in range(N_BLK):
        w_ref[b * BLK:(b + 1) * BLK, b * BLK:(b + 1) * BLK] = blocks_ref[0, b]
    y_ref[...] = jnp.dot(x_ref[...], w_ref[...],
                         preferred_element_type=jnp.float32)


def _transform(x, blocks, *, interpret=False):
    return pl.pallas_call(
        _transform_body,
        grid=(M_TILES, N_REL),
        in_specs=[
            pl.BlockSpec((M_TILE, DIM), lambda m, r: (m, 0)),
            pl.BlockSpec((1, N_BLK, BLK, BLK), lambda m, r: (r, 0, 0, 0)),
        ],
        out_specs=pl.BlockSpec((M_TILE, DIM), lambda m, r: (r * M_TILES + m, 0)),
        out_shape=jax.ShapeDtypeStruct((N_REL * N_NODES, DIM), jnp.float32),
        scratch_shapes=[pltpu.VMEM((DIM, DIM), jnp.float32)],
        interpret=interpret,
    )(x, blocks)


def _sc_body(y_hbm, gidx_hbm, tgt_hbm, zeros_hbm, out_hbm,
             idx_v, tgt_v, rows_v, acc_sh, sems, tsems):
    cid = lax.axis_index("c")
    sid = lax.axis_index("s")
    wid = cid * NS + sid

    zrows = N_NODES // 10
    row0 = sid * zrows

    @pl.when(sid < 10)
    def _():
        pltpu.sync_copy(zeros_hbm.at[pl.ds(row0, zrows)],
                        acc_sh.at[pl.ds(row0, zrows)])

    pltpu.sync_copy(gidx_hbm.at[wid], idx_v)
    plsc.subcore_barrier()

    def fire(b, c):
        pltpu.async_copy(tgt_hbm.at[wid * NCHUNKS + c], tgt_v.at[b], tsems[b])
        pltpu.async_copy(y_hbm.at[idx_v.at[c]], rows_v.at[b], sems[b])

    def drain(b, c):
        pltpu.make_async_copy(tgt_hbm.at[wid * NCHUNKS + c], tgt_v.at[b],
                              tsems[b]).wait()
        pltpu.make_async_copy(y_hbm.at[idx_v.at[c]], rows_v.at[b],
                              sems[b]).wait()
        pltpu.sync_copy(rows_v.at[b], acc_sh.at[tgt_v.at[b]], add=True)

    for b in range(NBUF):
        fire(b, b)

    def group(g, _):
        for b in range(NBUF):
            c = g * NBUF + b
            drain(b, c)
            nc = c + NBUF

            @pl.when(nc < NCHUNKS)
            def _():
                fire(b, nc)

        return ()

    lax.fori_loop(0, NCHUNKS // NBUF, group, ())
    for c in range(MAIN, NCHUNKS):
        drain(c % NBUF, c)

    plsc.subcore_barrier()

    @pl.when(sid < 10)
    def _():
        pltpu.sync_copy(acc_sh.at[pl.ds(row0, zrows)],
                        out_hbm.at[cid, pl.ds(row0, zrows)])


def _sc_scatter(y, gidx, target, zeros):
    mesh = plsc.VectorSubcoreMesh(core_axis_name="c", subcore_axis_name="s")
    k = functools.partial(
        pl.kernel,
        out_type=jax.ShapeDtypeStruct((NC, N_NODES, DIM), jnp.float32),
        mesh=mesh,
        scratch_types=[
            pltpu.VMEM((NCHUNKS, CHUNK), jnp.int32),
            pltpu.VMEM((NBUF, CHUNK), jnp.int32),
            pltpu.VMEM((NBUF, CHUNK, DIM), jnp.float32),
            pltpu.VMEM_SHARED((N_NODES, DIM), jnp.float32),
            [pltpu.SemaphoreType.DMA] * NBUF,
            [pltpu.SemaphoreType.DMA] * NBUF,
        ],
    )(_sc_body)
    return k(y, gidx.reshape(NW, NCHUNKS, CHUNK),
             target.reshape(NW * NCHUNKS, CHUNK), zeros)


def _combine_body(p_ref, o_ref):
    o_ref[...] = p_ref[0] + p_ref[1]


def _combine(partials, *, interpret=False):
    return pl.pallas_call(
        _combine_body,
        grid=(M_TILES,),
        in_specs=[pl.BlockSpec((NC, M_TILE, DIM), lambda i: (0, i, 0))],
        out_specs=pl.BlockSpec((M_TILE, DIM), lambda i: (i, 0)),
        out_shape=jax.ShapeDtypeStruct((N_NODES, DIM), jnp.float32),
        interpret=interpret,
    )(partials)


def kernel(x, source, target, edge_type, blocks):
    y = _transform(x, blocks)
    gidx = edge_type * N_NODES + source
    zeros = jnp.zeros((N_NODES, DIM), jnp.float32)
    partials = _sc_scatter(y, gidx, target, zeros)
    return _combine(partials)


# speedup vs baseline: 10.3058x; 1.1700x over previous
import functools

import jax
import jax.numpy as jnp
from jax import lax
from jax.experimental import pallas as pl
from jax.experimental.pallas import tpu as pltpu
from jax.experimental.pallas import tpu_sc as plsc

N_NODES = 10000
N_EDGES = 320000
N_REL = 24
DIM = 128
N_BLK = 8
BLK = 16

M_TILE = 5000
M_TILES = N_NODES // M_TILE

NC = 2
NS = 16
NW = NC * NS
EPW = N_EDGES // NW
CHUNK = 80
NCHUNKS = EPW // CHUNK
NBUF = 2
MAIN = (NCHUNKS // NBUF) * NBUF


def _transform_body(x_ref, blocks_ref, y_ref, w_ref):
    m = pl.program_id(0)
    r = pl.program_id(1)

    @pl.when((m == 0) & (r == 0))
    def _():
        w_ref[...] = jnp.zeros_like(w_ref)

    for b in range(N_BLK):
        w_ref[b * BLK:(b + 1) * BLK, b * BLK:(b + 1) * BLK] = blocks_ref[0, b]
    y_ref[...] = jnp.dot(x_ref[...], w_ref[...],
                         preferred_element_type=jnp.float32)


def _transform(x, blocks, *, interpret=False):
    return pl.pallas_call(
        _transform_body,
        grid=(M_TILES, N_REL),
        in_specs=[
            pl.BlockSpec((M_TILE, DIM), lambda m, r: (m, 0)),
            pl.BlockSpec((1, N_BLK, BLK, BLK), lambda m, r: (r, 0, 0, 0)),
        ],
        out_specs=pl.BlockSpec((M_TILE, DIM), lambda m, r: (r * M_TILES + m, 0)),
        out_shape=jax.ShapeDtypeStruct((N_REL * N_NODES, DIM), jnp.float32),
        scratch_shapes=[pltpu.VMEM((DIM, DIM), jnp.bfloat16)],
        interpret=interpret,
    )(x, blocks)


def _sc_body(y_hbm, gidx_hbm, tgt_hbm, zeros_hbm, out_hbm,
             idx_v, tgt_v, rows_v, acc_sh, sems, tsems):
    cid = lax.axis_index("c")
    sid = lax.axis_index("s")
    wid = cid * NS + sid

    zrows = N_NODES // 10
    row0 = sid * zrows

    @pl.when(sid < 10)
    def _():
        pltpu.sync_copy(zeros_hbm.at[pl.ds(row0, zrows)],
                        acc_sh.at[pl.ds(row0, zrows)])

    pltpu.sync_copy(gidx_hbm.at[wid], idx_v)
    plsc.subcore_barrier()

    def fire(b, c):
        pltpu.async_copy(tgt_hbm.at[wid * NCHUNKS + c], tgt_v.at[b], tsems[b])
        pltpu.async_copy(y_hbm.at[idx_v.at[c]], rows_v.at[b], sems[b])

    def drain(b, c):
        pltpu.make_async_copy(tgt_hbm.at[wid * NCHUNKS + c], tgt_v.at[b],
                              tsems[b]).wait()
        pltpu.make_async_copy(y_hbm.at[idx_v.at[c]], rows_v.at[b],
                              sems[b]).wait()
        pltpu.sync_copy(rows_v.at[b], acc_sh.at[tgt_v.at[b]], add=True)

    for b in range(NBUF):
        fire(b, b)

    def group(g, _):
        for b in range(NBUF):
            c = g * NBUF + b
            drain(b, c)
            nc = c + NBUF

            @pl.when(nc < NCHUNKS)
            def _():
                fire(b, nc)

        return ()

    lax.fori_loop(0, NCHUNKS // NBUF, group, ())
    for c in range(MAIN, NCHUNKS):
        drain(c % NBUF, c)

    plsc.subcore_barrier()

    @pl.when(sid < 10)
    def _():
        pltpu.sync_copy(acc_sh.at[pl.ds(row0, zrows)],
                        out_hbm.at[cid, pl.ds(row0, zrows)])


def _sc_scatter(y, gidx, target, zeros):
    mesh = plsc.VectorSubcoreMesh(core_axis_name="c", subcore_axis_name="s")
    k = functools.partial(
        pl.kernel,
        out_type=jax.ShapeDtypeStruct((NC, N_NODES, DIM), jnp.float32),
        mesh=mesh,
        scratch_types=[
            pltpu.VMEM((NCHUNKS, CHUNK), jnp.int32),
            pltpu.VMEM((NBUF, CHUNK), jnp.int32),
            pltpu.VMEM((NBUF, CHUNK, DIM), jnp.float32),
            pltpu.VMEM_SHARED((N_NODES, DIM), jnp.float32),
            [pltpu.SemaphoreType.DMA] * NBUF,
            [pltpu.SemaphoreType.DMA] * NBUF,
        ],
    )(_sc_body)
    return k(y, gidx.reshape(NW, NCHUNKS, CHUNK),
             target.reshape(NW * NCHUNKS, CHUNK), zeros)


def _combine_body(p_ref, o_ref):
    o_ref[...] = p_ref[0] + p_ref[1]


def _combine(partials, *, interpret=False):
    return pl.pallas_call(
        _combine_body,
        grid=(M_TILES,),
        in_specs=[pl.BlockSpec((NC, M_TILE, DIM), lambda i: (0, i, 0))],
        out_specs=pl.BlockSpec((M_TILE, DIM), lambda i: (i, 0)),
        out_shape=jax.ShapeDtypeStruct((N_NODES, DIM), jnp.float32),
        interpret=interpret,
    )(partials)


def kernel(x, source, target, edge_type, blocks):
    y = _transform(x.astype(jnp.bfloat16), blocks.astype(jnp.bfloat16))
    gidx = edge_type * N_NODES + source
    zeros = jnp.zeros((N_NODES, DIM), jnp.float32)
    partials = _sc_scatter(y, gidx, target, zeros)
    return _combine(partials)


# speedup vs baseline: 10.9022x; 1.0579x over previous
import functools

import jax
import jax.numpy as jnp
from jax import lax
from jax.experimental import pallas as pl
from jax.experimental.pallas import tpu as pltpu
from jax.experimental.pallas import tpu_sc as plsc

N_NODES = 10000
N_EDGES = 320000
N_REL = 24
DIM = 128
N_BLK = 8
BLK = 16

M_TILE = 10000
M_TILES = N_NODES // M_TILE

NC = 2
NS = 16
NW = NC * NS
EPW = N_EDGES // NW
CHUNK = 80
NCHUNKS = EPW // CHUNK
NBUF = 2
MAIN = (NCHUNKS // NBUF) * NBUF


def _transform_body(x_ref, blocks_ref, y_ref, w_ref):
    m = pl.program_id(0)
    r = pl.program_id(1)

    @pl.when((m == 0) & (r == 0))
    def _():
        w_ref[...] = jnp.zeros_like(w_ref)

    for b in range(N_BLK):
        w_ref[b * BLK:(b + 1) * BLK, b * BLK:(b + 1) * BLK] = blocks_ref[0, b]
    y_ref[...] = jnp.dot(x_ref[...], w_ref[...],
                         preferred_element_type=jnp.float32)


def _transform(x, blocks, *, interpret=False):
    return pl.pallas_call(
        _transform_body,
        grid=(M_TILES, N_REL),
        in_specs=[
            pl.BlockSpec((M_TILE, DIM), lambda m, r: (m, 0)),
            pl.BlockSpec((1, N_BLK, BLK, BLK), lambda m, r: (r, 0, 0, 0)),
        ],
        out_specs=pl.BlockSpec((M_TILE, DIM), lambda m, r: (r * M_TILES + m, 0)),
        out_shape=jax.ShapeDtypeStruct((N_REL * N_NODES, DIM), jnp.float32),
        scratch_shapes=[pltpu.VMEM((DIM, DIM), jnp.bfloat16)],
        interpret=interpret,
    )(x, blocks)


def _sc_body(y_hbm, gidx_hbm, tgt_hbm, zeros_hbm, out_hbm,
             idx_v, tgt_v, rows_v, acc_sh, sems, tsems):
    cid = lax.axis_index("c")
    sid = lax.axis_index("s")
    wid = cid * NS + sid

    zrows = N_NODES // 10
    row0 = sid * zrows

    @pl.when(sid < 10)
    def _():
        pltpu.sync_copy(zeros_hbm.at[pl.ds(row0, zrows)],
                        acc_sh.at[pl.ds(row0, zrows)])

    pltpu.sync_copy(gidx_hbm.at[wid], idx_v)
    plsc.subcore_barrier()

    def fire(b, c):
        pltpu.async_copy(tgt_hbm.at[wid * NCHUNKS + c], tgt_v.at[b], tsems[b])
        pltpu.async_copy(y_hbm.at[idx_v.at[c]], rows_v.at[b], sems[b])

    def drain(b, c):
        pltpu.make_async_copy(tgt_hbm.at[wid * NCHUNKS + c], tgt_v.at[b],
                              tsems[b]).wait()
        pltpu.make_async_copy(y_hbm.at[idx_v.at[c]], rows_v.at[b],
                              sems[b]).wait()
        pltpu.sync_copy(rows_v.at[b], acc_sh.at[tgt_v.at[b]], add=True)

    for b in range(NBUF):
        fire(b, b)

    def group(g, _):
        for b in range(NBUF):
            c = g * NBUF + b
            drain(b, c)
            nc = c + NBUF

            @pl.when(nc < NCHUNKS)
            def _():
                fire(b, nc)

        return ()

    lax.fori_loop(0, NCHUNKS // NBUF, group, ())
    for c in range(MAIN, NCHUNKS):
        drain(c % NBUF, c)

    plsc.subcore_barrier()

    @pl.when(sid < 10)
    def _():
        pltpu.sync_copy(acc_sh.at[pl.ds(row0, zrows)],
                        out_hbm.at[cid, pl.ds(row0, zrows)])


def _sc_scatter(y, gidx, target, zeros):
    mesh = plsc.VectorSubcoreMesh(core_axis_name="c", subcore_axis_name="s")
    k = functools.partial(
        pl.kernel,
        out_type=jax.ShapeDtypeStruct((NC, N_NODES, DIM), jnp.float32),
        mesh=mesh,
        scratch_types=[
            pltpu.VMEM((NCHUNKS, CHUNK), jnp.int32),
            pltpu.VMEM((NBUF, CHUNK), jnp.int32),
            pltpu.VMEM((NBUF, CHUNK, DIM), jnp.float32),
            pltpu.VMEM_SHARED((N_NODES, DIM), jnp.float32),
            [pltpu.SemaphoreType.DMA] * NBUF,
            [pltpu.SemaphoreType.DMA] * NBUF,
        ],
    )(_sc_body)
    return k(y, gidx.reshape(NW, NCHUNKS, CHUNK),
             target.reshape(NW * NCHUNKS, CHUNK), zeros)


def _combine_body(p_ref, o_ref):
    o_ref[...] = p_ref[0] + p_ref[1]


def _combine(partials, *, interpret=False):
    return pl.pallas_call(
        _combine_body,
        grid=(M_TILES,),
        in_specs=[pl.BlockSpec((NC, M_TILE, DIM), lambda i: (0, i, 0))],
        out_specs=pl.BlockSpec((M_TILE, DIM), lambda i: (i, 0)),
        out_shape=jax.ShapeDtypeStruct((N_NODES, DIM), jnp.float32),
        interpret=interpret,
    )(partials)


def kernel(x, source, target, edge_type, blocks):
    y = _transform(x.astype(jnp.bfloat16), blocks.astype(jnp.bfloat16))
    gidx = edge_type * N_NODES + source
    zeros = jnp.zeros((N_NODES, DIM), jnp.float32)
    partials = _sc_scatter(y, gidx, target, zeros)
    return _combine(partials)


# speedup vs baseline: 11.3962x; 1.0453x over previous
import functools

import jax
import jax.numpy as jnp
from jax import lax
from jax.experimental import pallas as pl
from jax.experimental.pallas import tpu as pltpu
from jax.experimental.pallas import tpu_sc as plsc

N_NODES = 10000
N_EDGES = 320000
N_REL = 24
DIM = 128
N_BLK = 8
BLK = 16

M_TILE = 10000
M_TILES = N_NODES // M_TILE

NC = 2
NS = 16
NW = NC * NS
EPW = N_EDGES // NW
CHUNK = 80
NCHUNKS = EPW // CHUNK
NBUF = 2
MAIN = (NCHUNKS // NBUF) * NBUF


def _transform_body(x_ref, blocks_ref, y_ref, xb_ref, w_ref):
    m = pl.program_id(0)
    r = pl.program_id(1)

    @pl.when((m == 0) & (r == 0))
    def _():
        w_ref[...] = jnp.zeros_like(w_ref)
        xb_ref[...] = x_ref[...].astype(jnp.bfloat16)

    for b in range(N_BLK):
        w_ref[b * BLK:(b + 1) * BLK, b * BLK:(b + 1) * BLK] = (
            blocks_ref[0, b].astype(jnp.bfloat16))
    y_ref[...] = jnp.dot(xb_ref[...], w_ref[...],
                         preferred_element_type=jnp.float32)


def _transform(x, blocks, *, interpret=False):
    return pl.pallas_call(
        _transform_body,
        grid=(M_TILES, N_REL),
        in_specs=[
            pl.BlockSpec((M_TILE, DIM), lambda m, r: (m, 0)),
            pl.BlockSpec((1, N_BLK, BLK, BLK), lambda m, r: (r, 0, 0, 0)),
        ],
        out_specs=pl.BlockSpec((M_TILE, DIM), lambda m, r: (r * M_TILES + m, 0)),
        out_shape=jax.ShapeDtypeStruct((N_REL * N_NODES, DIM), jnp.float32),
        scratch_shapes=[pltpu.VMEM((M_TILE, DIM), jnp.bfloat16),
                        pltpu.VMEM((DIM, DIM), jnp.bfloat16)],
        interpret=interpret,
    )(x, blocks)


def _sc_body(y_hbm, gidx_hbm, tgt_hbm, zeros_hbm, out_hbm,
             idx_v, tgt_v, rows_v, acc_sh, sems, tsems):
    cid = lax.axis_index("c")
    sid = lax.axis_index("s")
    wid = cid * NS + sid

    zrows = N_NODES // 10
    row0 = sid * zrows

    @pl.when(sid < 10)
    def _():
        pltpu.sync_copy(zeros_hbm, acc_sh.at[pl.ds(row0, zrows)])

    pltpu.sync_copy(gidx_hbm.at[pl.ds(wid * EPW, EPW)], idx_v)
    plsc.subcore_barrier()

    def fire(b, c):
        pltpu.async_copy(tgt_hbm.at[pl.ds(wid * EPW + c * CHUNK, CHUNK)],
                         tgt_v.at[b], tsems[b])
        pltpu.async_copy(y_hbm.at[idx_v.at[pl.ds(c * CHUNK, CHUNK)]],
                         rows_v.at[b], sems[b])

    def drain(b, c):
        pltpu.make_async_copy(tgt_hbm.at[pl.ds(wid * EPW + c * CHUNK, CHUNK)],
                              tgt_v.at[b], tsems[b]).wait()
        pltpu.make_async_copy(y_hbm.at[idx_v.at[pl.ds(c * CHUNK, CHUNK)]],
                              rows_v.at[b], sems[b]).wait()
        pltpu.sync_copy(rows_v.at[b], acc_sh.at[tgt_v.at[b]], add=True)

    for b in range(NBUF):
        fire(b, b)

    def group(g, _):
        for b in range(NBUF):
            c = g * NBUF + b
            drain(b, c)
            nc = c + NBUF

            @pl.when(nc < NCHUNKS)
            def _():
                fire(b, nc)

        return ()

    lax.fori_loop(0, NCHUNKS // NBUF, group, ())
    for c in range(MAIN, NCHUNKS):
        drain(c % NBUF, c)

    plsc.subcore_barrier()

    @pl.when(sid < 10)
    def _():
        pltpu.sync_copy(acc_sh.at[pl.ds(row0, zrows)],
                        out_hbm.at[cid, pl.ds(row0, zrows)])


def _sc_scatter(y, gidx, target, zeros):
    mesh = plsc.VectorSubcoreMesh(core_axis_name="c", subcore_axis_name="s")
    k = functools.partial(
        pl.kernel,
        out_type=jax.ShapeDtypeStruct((NC, N_NODES, DIM), jnp.float32),
        mesh=mesh,
        scratch_types=[
            pltpu.VMEM((EPW,), jnp.int32),
            pltpu.VMEM((NBUF, CHUNK), jnp.int32),
            pltpu.VMEM((NBUF, CHUNK, DIM), jnp.float32),
            pltpu.VMEM_SHARED((N_NODES, DIM), jnp.float32),
            [pltpu.SemaphoreType.DMA] * NBUF,
            [pltpu.SemaphoreType.DMA] * NBUF,
        ],
    )(_sc_body)
    return k(y, gidx, target, zeros)


def _combine_body(p_ref, o_ref):
    o_ref[...] = p_ref[0] + p_ref[1]


def _combine(partials, *, interpret=False):
    return pl.pallas_call(
        _combine_body,
        grid=(M_TILES,),
        in_specs=[pl.BlockSpec((NC, M_TILE, DIM), lambda i: (0, i, 0))],
        out_specs=pl.BlockSpec((M_TILE, DIM), lambda i: (i, 0)),
        out_shape=jax.ShapeDtypeStruct((N_NODES, DIM), jnp.float32),
        interpret=interpret,
    )(partials)


def kernel(x, source, target, edge_type, blocks):
    y = _transform(x, blocks)
    gidx = edge_type * N_NODES + source
    zeros = jnp.zeros((N_NODES // 10, DIM), jnp.float32)
    partials = _sc_scatter(y, gidx, target, zeros)
    return _combine(partials)


# speedup vs baseline: 12.8622x; 1.1286x over previous
import functools

import jax
import jax.numpy as jnp
from jax import lax
from jax.experimental import pallas as pl
from jax.experimental.pallas import tpu as pltpu
from jax.experimental.pallas import tpu_sc as plsc

N_NODES = 10000
N_EDGES = 320000
N_REL = 24
DIM = 128
N_BLK = 8
BLK = 16

M_TILE = 10000
M_TILES = N_NODES // M_TILE

NC = 2
NS = 16
NW = NC * NS
EPW = N_EDGES // NW
CHUNK = 80
NCHUNKS = EPW // CHUNK
NBUF = 3
MAIN = (NCHUNKS // NBUF) * NBUF


def _transform_body(x_ref, blocks_ref, y_ref, xb_ref, w_ref):
    m = pl.program_id(0)
    r = pl.program_id(1)

    @pl.when((m == 0) & (r == 0))
    def _():
        w_ref[...] = jnp.zeros_like(w_ref)
        xb_ref[...] = x_ref[...].astype(jnp.bfloat16)

    for b in range(N_BLK):
        w_ref[b * BLK:(b + 1) * BLK, b * BLK:(b + 1) * BLK] = (
            blocks_ref[0, b].astype(jnp.bfloat16))
    y_ref[...] = jnp.dot(xb_ref[...], w_ref[...],
                         preferred_element_type=jnp.float32)


def _transform(x, blocks, *, interpret=False):
    return pl.pallas_call(
        _transform_body,
        grid=(M_TILES, N_REL),
        in_specs=[
            pl.BlockSpec((M_TILE, DIM), lambda m, r: (m, 0)),
            pl.BlockSpec((1, N_BLK, BLK, BLK), lambda m, r: (r, 0, 0, 0)),
        ],
        out_specs=pl.BlockSpec((M_TILE, DIM), lambda m, r: (r * M_TILES + m, 0)),
        out_shape=jax.ShapeDtypeStruct((N_REL * N_NODES, DIM), jnp.float32),
        scratch_shapes=[pltpu.VMEM((M_TILE, DIM), jnp.bfloat16),
                        pltpu.VMEM((DIM, DIM), jnp.bfloat16)],
        interpret=interpret,
    )(x, blocks)


def _sc_body(y_hbm, gidx_hbm, tgt_hbm, zeros_hbm, out_hbm,
             idx_v, tgt_v, rows_v, acc_sh, sems, tsems):
    cid = lax.axis_index("c")
    sid = lax.axis_index("s")
    wid = cid * NS + sid

    zrows = N_NODES // 10
    row0 = sid * zrows

    @pl.when(sid < 10)
    def _():
        pltpu.sync_copy(zeros_hbm, acc_sh.at[pl.ds(row0, zrows)])

    pltpu.sync_copy(gidx_hbm.at[pl.ds(wid * EPW, EPW)], idx_v)
    plsc.subcore_barrier()

    def fire(b, c):
        pltpu.async_copy(tgt_hbm.at[pl.ds(wid * EPW + c * CHUNK, CHUNK)],
                         tgt_v.at[b], tsems[b])
        pltpu.async_copy(y_hbm.at[idx_v.at[pl.ds(c * CHUNK, CHUNK)]],
                         rows_v.at[b], sems[b])

    def drain(b, c):
        pltpu.make_async_copy(tgt_hbm.at[pl.ds(wid * EPW + c * CHUNK, CHUNK)],
                              tgt_v.at[b], tsems[b]).wait()
        pltpu.make_async_copy(y_hbm.at[idx_v.at[pl.ds(c * CHUNK, CHUNK)]],
                              rows_v.at[b], sems[b]).wait()
        pltpu.sync_copy(rows_v.at[b], acc_sh.at[tgt_v.at[b]], add=True)

    for b in range(NBUF):
        fire(b, b)

    def group(g, _):
        for b in range(NBUF):
            c = g * NBUF + b
            drain(b, c)
            nc = c + NBUF

            @pl.when(nc < NCHUNKS)
            def _():
                fire(b, nc)

        return ()

    lax.fori_loop(0, NCHUNKS // NBUF, group, ())
    for c in range(MAIN, NCHUNKS):
        drain(c % NBUF, c)

    plsc.subcore_barrier()

    @pl.when(sid < 10)
    def _():
        pltpu.sync_copy(acc_sh.at[pl.ds(row0, zrows)],
                        out_hbm.at[cid, pl.ds(row0, zrows)])


def _sc_scatter(y, gidx, target, zeros):
    mesh = plsc.VectorSubcoreMesh(core_axis_name="c", subcore_axis_name="s")
    k = functools.partial(
        pl.kernel,
        out_type=jax.ShapeDtypeStruct((NC, N_NODES, DIM), jnp.float32),
        mesh=mesh,
        scratch_types=[
            pltpu.VMEM((EPW,), jnp.int32),
            pltpu.VMEM((NBUF, CHUNK), jnp.int32),
            pltpu.VMEM((NBUF, CHUNK, DIM), jnp.float32),
            pltpu.VMEM_SHARED((N_NODES, DIM), jnp.float32),
            [pltpu.SemaphoreType.DMA] * NBUF,
            [pltpu.SemaphoreType.DMA] * NBUF,
        ],
    )(_sc_body)
    return k(y, gidx, target, zeros)


def _combine_body(p_ref, o_ref):
    o_ref[...] = p_ref[0] + p_ref[1]


def _combine(partials, *, interpret=False):
    return pl.pallas_call(
        _combine_body,
        grid=(M_TILES,),
        in_specs=[pl.BlockSpec((NC, M_TILE, DIM), lambda i: (0, i, 0))],
        out_specs=pl.BlockSpec((M_TILE, DIM), lambda i: (i, 0)),
        out_shape=jax.ShapeDtypeStruct((N_NODES, DIM), jnp.float32),
        interpret=interpret,
    )(partials)


def kernel(x, source, target, edge_type, blocks):
    y = _transform(x, blocks)
    gidx = edge_type * N_NODES + source
    zeros = jnp.zeros((N_NODES // 10, DIM), jnp.float32)
    partials = _sc_scatter(y, gidx, target, zeros)
    return _combine(partials)
